# baseline (speedup 1.0000x reference)
import functools

import numpy as np

import jax
import jax.numpy as jnp
from jax import lax
from jax.experimental import pallas as pl
from jax.experimental.pallas import tpu as pltpu
from jax.experimental.pallas import tpu_sc as plsc


def _make_tc_pass(n, c, kb, nstream):
    ncb = c // (kb * nstream)

    def body(*refs):
        x_refs = refs[:nstream]
        tgt_ref = refs[nstream]
        lse_ref, tval_ref = refs[nstream + 1], refs[nstream + 2]
        m_ref, s_ref, tv_ref = refs[nstream + 3:]
        j = pl.program_id(0)

        @pl.when(j == 0)
        def _init():
            m_ref[...] = jnp.full((1, n), -jnp.inf, jnp.float32)
            s_ref[...] = jnp.zeros((1, n), jnp.float32)
            tv_ref[...] = jnp.zeros((1, n), jnp.float32)

        xs = [r[...] for r in x_refs]
        m_new = m_ref[...]
        for x in xs:
            m_new = jnp.maximum(m_new, jnp.max(x, axis=0, keepdims=True))
        s_new = s_ref[...] * jnp.exp(m_ref[...] - m_new)
        tv_new = tv_ref[...]
        iota = lax.broadcasted_iota(jnp.int32, (kb, n), 0)
        for t, x in enumerate(xs):
            s_new = s_new + jnp.sum(jnp.exp(x - m_new), axis=0, keepdims=True)
            eq = iota == tgt_ref[...] - (j * nstream + t) * kb
            tv_new = tv_new + jnp.sum(
                jnp.where(eq, x, 0.0), axis=0, keepdims=True
            )
        m_ref[...] = m_new
        s_ref[...] = s_new
        tv_ref[...] = tv_new

        @pl.when(j == ncb - 1)
        def _fin():
            lse_ref[...] = m_ref[...] + jnp.log(s_ref[...])
            tval_ref[...] = tv_ref[...]

    def x_spec(t):
        return pl.BlockSpec((kb, n), lambda j, t=t: (j * nstream + t, 0))

    return pl.pallas_call(
        body,
        grid=(ncb,),
        in_specs=[x_spec(t) for t in range(nstream)]
        + [pl.BlockSpec((1, n), lambda j: (0, 0))],
        out_specs=[
            pl.BlockSpec((1, n), lambda j: (0, 0)),
            pl.BlockSpec((1, n), lambda j: (0, 0)),
        ],
        out_shape=[
            jax.ShapeDtypeStruct((1, n), jnp.float32),
            jax.ShapeDtypeStruct((1, n), jnp.float32),
        ],
        scratch_shapes=[
            pltpu.VMEM((1, n), jnp.float32),
            pltpu.VMEM((1, n), jnp.float32),
            pltpu.VMEM((1, n), jnp.float32),
        ],
    )


def _make_sc_combine(n):
    info = plsc.get_sparse_core_info()
    lanes = info.num_lanes
    mesh = plsc.VectorSubcoreMesh(core_axis_name="c", subcore_axis_name="s")

    @functools.partial(
        pl.kernel,
        mesh=mesh,
        out_type=jax.ShapeDtypeStruct((lanes,), jnp.float32),
        scratch_types=[
            pltpu.VMEM((n,), jnp.float32),
            pltpu.VMEM((n,), jnp.float32),
            pltpu.VMEM((n,), jnp.float32),
            pltpu.VMEM((lanes,), jnp.float32),
        ],
    )
    def sc_combine(ratio_hbm, tval_hbm, lse_hbm, out_hbm, r_v, t_v, l_v, o_v):
        wid = lax.axis_index("s") * info.num_cores + lax.axis_index("c")

        @pl.when(wid == 0)
        def _():
            pltpu.sync_copy(ratio_hbm, r_v)
            pltpu.sync_copy(tval_hbm, t_v)
            pltpu.sync_copy(lse_hbm, l_v)
            acc = jnp.zeros((lanes,), jnp.float32)
            for k in range(n // lanes):
                sl = pl.ds(k * lanes, lanes)
                acc = acc + r_v[sl] * (t_v[sl] - l_v[sl])
            lane_ids = lax.broadcasted_iota(jnp.int32, (lanes,), 0)
            dnums = lax.GatherDimensionNumbers(
                offset_dims=(),
                collapsed_slice_dims=(0,),
                start_index_map=(0,),
            )
            step = 1
            while step < lanes:
                perm = (lane_ids ^ step).reshape(lanes, 1)
                acc = acc + lax.gather(
                    acc,
                    perm,
                    dnums,
                    (1,),
                    mode=lax.GatherScatterMode.PROMISE_IN_BOUNDS,
                )
                step *= 2
            o_v[...] = acc * (-1.0 / n)
            pltpu.sync_copy(o_v, out_hbm)

    return sc_combine


@jax.jit
def kernel(ratio, inputs, targets):
    n, c = inputs.shape
    xt = jnp.swapaxes(inputs, 0, 1)
    tgt = targets.astype(jnp.int32).reshape(1, n)
    nstream = 5
    lse, tval = _make_tc_pass(n, c, kb=1000, nstream=nstream)(
        *([xt] * nstream), tgt
    )
    out = _make_sc_combine(n)(
        ratio.reshape(n), tval.reshape(n), lse.reshape(n)
    )
    return out[0]


# speedup vs baseline: 1.0516x; 1.0516x over previous
import functools

import numpy as np

import jax
import jax.numpy as jnp
from jax import lax
from jax.experimental import pallas as pl
from jax.experimental.pallas import tpu as pltpu
from jax.experimental.pallas import tpu_sc as plsc


def _make_tc_pass(n, c, kb, nstream):
    ncb = c // (kb * nstream)

    def body(*refs):
        x_refs = refs[:nstream]
        tgt_ref = refs[nstream]
        lse_ref, tval_ref = refs[nstream + 1], refs[nstream + 2]
        m_ref, s_ref, tv_ref = refs[nstream + 3:]
        j = pl.program_id(0)

        @pl.when(j == 0)
        def _init():
            m_ref[...] = jnp.full((1, n), -jnp.inf, jnp.float32)
            s_ref[...] = jnp.zeros((1, n), jnp.float32)
            tv_ref[...] = jnp.zeros((1, n), jnp.float32)

        xs = [r[...] for r in x_refs]
        m_new = m_ref[...]
        for x in xs:
            m_new = jnp.maximum(m_new, jnp.max(x, axis=0, keepdims=True))
        s_new = s_ref[...] * jnp.exp(m_ref[...] - m_new)
        tv_new = tv_ref[...]
        iota = lax.broadcasted_iota(jnp.int32, (kb, n), 0)
        for t, x in enumerate(xs):
            s_new = s_new + jnp.sum(jnp.exp(x - m_new), axis=0, keepdims=True)
            eq = iota == tgt_ref[...] - (j * nstream + t) * kb
            tv_new = tv_new + jnp.sum(
                jnp.where(eq, x, 0.0), axis=0, keepdims=True
            )
        m_ref[...] = m_new
        s_ref[...] = s_new
        tv_ref[...] = tv_new

        @pl.when(j == ncb - 1)
        def _fin():
            lse_ref[...] = m_ref[...] + jnp.log(s_ref[...])
            tval_ref[...] = tv_ref[...]

    def x_spec(t):
        return pl.BlockSpec((kb, n), lambda j, t=t: (j * nstream + t, 0))

    return pl.pallas_call(
        body,
        grid=(ncb,),
        in_specs=[x_spec(t) for t in range(nstream)]
        + [pl.BlockSpec((1, n), lambda j: (0, 0))],
        out_specs=[
            pl.BlockSpec((1, n), lambda j: (0, 0)),
            pl.BlockSpec((1, n), lambda j: (0, 0)),
        ],
        out_shape=[
            jax.ShapeDtypeStruct((1, n), jnp.float32),
            jax.ShapeDtypeStruct((1, n), jnp.float32),
        ],
        scratch_shapes=[
            pltpu.VMEM((1, n), jnp.float32),
            pltpu.VMEM((1, n), jnp.float32),
            pltpu.VMEM((1, n), jnp.float32),
        ],
    )


def _make_sc_combine(n):
    info = plsc.get_sparse_core_info()
    lanes = info.num_lanes
    mesh = plsc.VectorSubcoreMesh(
        core_axis_name="c", subcore_axis_name="s", num_cores=1
    )

    @functools.partial(
        pl.kernel,
        mesh=mesh,
        out_type=jax.ShapeDtypeStruct((lanes,), jnp.float32),
        scratch_types=[
            pltpu.VMEM((n,), jnp.float32),
            pltpu.VMEM((n,), jnp.float32),
            pltpu.VMEM((n,), jnp.float32),
            pltpu.VMEM((lanes,), jnp.float32),
        ],
    )
    def sc_combine(ratio_hbm, tval_hbm, lse_hbm, out_hbm, r_v, t_v, l_v, o_v):
        wid = lax.axis_index("s") * info.num_cores + lax.axis_index("c")

        @pl.when(wid == 0)
        def _():
            pltpu.sync_copy(ratio_hbm, r_v)
            pltpu.sync_copy(tval_hbm, t_v)
            pltpu.sync_copy(lse_hbm, l_v)
            acc = jnp.zeros((lanes,), jnp.float32)
            for k in range(n // lanes):
                sl = pl.ds(k * lanes, lanes)
                acc = acc + r_v[sl] * (t_v[sl] - l_v[sl])
            lane_ids = lax.broadcasted_iota(jnp.int32, (lanes,), 0)
            dnums = lax.GatherDimensionNumbers(
                offset_dims=(),
                collapsed_slice_dims=(0,),
                start_index_map=(0,),
            )
            step = 1
            while step < lanes:
                perm = (lane_ids ^ step).reshape(lanes, 1)
                acc = acc + lax.gather(
                    acc,
                    perm,
                    dnums,
                    (1,),
                    mode=lax.GatherScatterMode.PROMISE_IN_BOUNDS,
                )
                step *= 2
            o_v[...] = acc * (-1.0 / n)
            pltpu.sync_copy(o_v, out_hbm)

    return sc_combine


@jax.jit
def kernel(ratio, inputs, targets):
    n, c = inputs.shape
    xt = jnp.swapaxes(inputs, 0, 1)
    tgt = targets.astype(jnp.int32).reshape(1, n)
    nstream = 4
    lse, tval = _make_tc_pass(n, c, kb=1000, nstream=nstream)(
        *([xt] * nstream), tgt
    )
    out = _make_sc_combine(n)(
        ratio.reshape(n), tval.reshape(n), lse.reshape(n)
    )
    return out[0]
